# baseline (speedup 1.0000x reference)
import functools

import jax
import jax.numpy as jnp
from jax import lax
from jax.experimental import pallas as pl
from jax.experimental.pallas import tpu as pltpu
from jax.experimental.pallas import tpu_sc as plsc

_THRESH = 0.7
_KEPT = 100000

_B, _C, _H, _W = 8, 19, 512, 512
_N = _B * _H * _W
_HT = 128
_RC = 8
_HTR = 128


_LOG2E = 1.4426950408889634
_LN2 = 0.6931471805599453
_CAP = 120.0


def _ce_fused_kernel(score_ref, target_ref, cnt_ref, sum_ref, loss_ref):
    acc_c = jnp.zeros((_RC, _W), jnp.float32)
    acc_s = jnp.zeros((_RC, _W), jnp.float32)
    for r in range(0, _HT, _RC):
        rs = pl.ds(r, _RC)
        t = target_ref[0, rs]
        y0 = jnp.minimum(score_ref[0, 0, rs] * _LOG2E, _CAP)
        y1 = jnp.minimum(score_ref[0, 1, rs] * _LOG2E, _CAP)
        s0 = jnp.exp2(y0)
        s1 = jnp.exp2(y1)
        g0 = jnp.where(t == 0, y0, 0.0)
        g1 = jnp.where(t == 1, y1, 0.0)
        for c in range(2, _C - 1, 2):
            ya = jnp.minimum(score_ref[0, c, rs] * _LOG2E, _CAP)
            yb = jnp.minimum(score_ref[0, c + 1, rs] * _LOG2E, _CAP)
            s0 = s0 + jnp.exp2(ya)
            s1 = s1 + jnp.exp2(yb)
            g0 = g0 + jnp.where(t == c, ya, 0.0)
            g1 = g1 + jnp.where(t == c + 1, yb, 0.0)
        yl = jnp.minimum(score_ref[0, _C - 1, rs] * _LOG2E, _CAP)
        s = s0 + s1 + jnp.exp2(yl)
        yt = g0 + g1 + jnp.where(t == _C - 1, yl, 0.0)
        log2p_t = yt - jnp.log2(s)
        keep = jnp.exp2(log2p_t) < _THRESH
        acc_c = acc_c + keep.astype(jnp.float32)
        acc_s = acc_s + jnp.where(keep, log2p_t, 0.0)
    cc = jnp.sum(acc_c)
    sm = jnp.sum(acc_s) * (-_LN2)

    @pl.when((pl.program_id(0) == 0) & (pl.program_id(1) == 0))
    def _():
        cnt_ref[...] = jnp.zeros((1, 1), jnp.float32)
        sum_ref[...] = jnp.zeros((1, 1), jnp.float32)

    cnt_ref[...] += cc
    sum_ref[...] += sm

    @pl.when((pl.program_id(0) == _B - 1) & (pl.program_id(1) == _H // _HT - 1))
    def _():
        loss_ref[...] = sum_ref[...] / jnp.maximum(cnt_ref[...], 1.0)


def _ce_arrays_kernel(score_ref, target_ref, pred_ref, nll_ref):
    x = score_ref[0]
    t = target_ref[0]
    m = jnp.max(x, axis=0)
    s = jnp.sum(jnp.exp(x - m[None]), axis=0)
    cls = lax.broadcasted_iota(jnp.int32, x.shape, 0)
    xt = jnp.sum(jnp.where(cls == t[None], x, 0.0), axis=0)
    logp_t = xt - m - jnp.log(s)
    pred_ref[0] = jnp.exp(logp_t)
    nll_ref[0] = -logp_t


_NW = 32
_CHUNK = _N // _NW
_SC_MESH = plsc.VectorSubcoreMesh(core_axis_name="c", subcore_axis_name="s")


def _make_sc_hist(match_shift, shift, nb):
    @functools.partial(
        pl.kernel,
        mesh=_SC_MESH,
        compiler_params=pltpu.CompilerParams(needs_layout_passes=False),
        out_type=jax.ShapeDtypeStruct((_NW, nb), jnp.int32),
        scratch_types=[
            pltpu.VMEM((_CHUNK,), jnp.float32),
            pltpu.VMEM((1, 128), jnp.int32),
            pltpu.VMEM((16, nb), jnp.int32),
            pltpu.VMEM((nb,), jnp.int32),
        ],
    )
    def sc_hist(pred_hbm, state_hbm, out_hbm, chunk_v, st_v, hist_v, loc_v):
        wid = lax.axis_index("s") * 2 + lax.axis_index("c")
        pltpu.sync_copy(pred_hbm.at[pl.ds(wid * _CHUNK, _CHUNK)], chunk_v)
        pltpu.sync_copy(state_hbm, st_v)
        pfx = st_v[0, pl.ds(0, 16)]
        zeros16 = jnp.zeros((16,), jnp.int32)

        def zero_body(j, _):
            for l in range(16):
                hist_v[l, pl.ds(j * 16, 16)] = zeros16
            return 0

        lax.fori_loop(0, nb // 16, zero_body, 0)

        lane = jnp.arange(16, dtype=jnp.int32)
        ones = jnp.ones((16,), jnp.int32)

        def hist_body(i, _):
            bits = lax.bitcast_convert_type(chunk_v[pl.ds(i * 16, 16)],
                                            jnp.int32)
            match = lax.shift_right_arithmetic(bits, match_shift) == pfx
            bin_ = lax.shift_right_arithmetic(bits, shift) & (nb - 1)
            plsc.addupdate_scatter(hist_v, [lane, bin_], ones, mask=match)
            return 0

        lax.fori_loop(0, _CHUNK // 16, hist_body, 0)

        def fold_body(j, _):
            acc = hist_v[0, pl.ds(j * 16, 16)]
            for l in range(1, 16):
                acc = acc + hist_v[l, pl.ds(j * 16, 16)]
            loc_v[pl.ds(j * 16, 16)] = acc
            return 0

        lax.fori_loop(0, nb // 16, fold_body, 0)
        pltpu.sync_copy(loc_v, out_hbm.at[wid])

    return sc_hist


def _make_locate(nb):
    def locate(hist_ref, state_ref, out_ref, thr_ref):
        tot = jnp.sum(hist_ref[...], axis=0).reshape(1, nb)
        cum = tot
        s = 1
        while s < nb:
            sh = jnp.concatenate(
                [jnp.zeros((1, s), jnp.int32), cum[:, : nb - s]], axis=1)
            cum = cum + sh
            s *= 2
        r = state_ref[0, 16]
        pfx = state_ref[0, 0]
        target = r + 1
        below = cum < target
        bstar = jnp.sum(below.astype(jnp.int32))
        cumex = jnp.sum(jnp.where(below, tot, 0))
        newpfx = pfx * nb + bstar
        newr = r - cumex
        lanes = lax.broadcasted_iota(jnp.int32, (1, 128), 1)
        out_ref[...] = jnp.where(lanes == 16, newr, newpfx)
        thr_ref[...] = lax.bitcast_convert_type(newpfx,
                                                jnp.float32).reshape(1, 1)

    return locate


def _sc_select(pred_flat):
    lanes = jnp.arange(128, dtype=jnp.int32).reshape(1, 128)
    state = jnp.where(lanes == 16, jnp.int32(_KEPT), 0)
    thr = None
    for match_shift, shift, nb in ((31, 20, 2048), (20, 10, 1024),
                                   (10, 0, 1024)):
        hists = _make_sc_hist(match_shift, shift, nb)(pred_flat, state)
        state, thr = pl.pallas_call(
            _make_locate(nb),
            in_specs=[
                pl.BlockSpec((_NW, nb), lambda: (0, 0)),
                pl.BlockSpec((1, 128), lambda: (0, 0)),
            ],
            out_specs=[
                pl.BlockSpec((1, 128), lambda: (0, 0)),
                pl.BlockSpec((1, 1), lambda: (0, 0)),
            ],
            out_shape=[
                jax.ShapeDtypeStruct((1, 128), jnp.int32),
                jax.ShapeDtypeStruct((1, 1), jnp.float32),
            ],
        )(hists, state)
    return thr


def _masked_sum_kernel(pred_ref, nll_ref, thr_ref, cnt_ref, sum_ref):
    thr = thr_ref[0, 0]
    keep = pred_ref[...] < thr
    c = jnp.sum(keep.astype(jnp.float32))
    sm = jnp.sum(jnp.where(keep, nll_ref[...], 0.0))

    @pl.when(pl.program_id(0) == 0)
    def _():
        cnt_ref[...] = jnp.zeros((1, 1), jnp.float32)
        sum_ref[...] = jnp.zeros((1, 1), jnp.float32)

    cnt_ref[...] += c
    sum_ref[...] += sm


def _rare_path(score, target):
    pred, nll = pl.pallas_call(
        _ce_arrays_kernel,
        grid=(_B, _H // _HTR),
        in_specs=[
            pl.BlockSpec((1, _C, _HTR, _W), lambda b, h: (b, 0, h, 0)),
            pl.BlockSpec((1, _HTR, _W), lambda b, h: (b, h, 0)),
        ],
        out_specs=[
            pl.BlockSpec((1, _HTR, _W), lambda b, h: (b, h, 0)),
            pl.BlockSpec((1, _HTR, _W), lambda b, h: (b, h, 0)),
        ],
        out_shape=[
            jax.ShapeDtypeStruct((_B, _H, _W), jnp.float32),
            jax.ShapeDtypeStruct((_B, _H, _W), jnp.float32),
        ],
    )(score, target)
    pred2 = pred.reshape(_N // 1024, 1024)
    nll2 = nll.reshape(_N // 1024, 1024)

    thr = _sc_select(pred.reshape(_N))

    rows = _N // 1024
    rt = rows // 8
    cnt, sm = pl.pallas_call(
        _masked_sum_kernel,
        grid=(8,),
        in_specs=[
            pl.BlockSpec((rt, 1024), lambda i: (i, 0)),
            pl.BlockSpec((rt, 1024), lambda i: (i, 0)),
            pl.BlockSpec((1, 1), lambda i: (0, 0)),
        ],
        out_specs=[
            pl.BlockSpec((1, 1), lambda i: (0, 0)),
            pl.BlockSpec((1, 1), lambda i: (0, 0)),
        ],
        out_shape=[
            jax.ShapeDtypeStruct((1, 1), jnp.float32),
            jax.ShapeDtypeStruct((1, 1), jnp.float32),
        ],
    )(pred2, nll2, thr)
    return sm[0, 0] / jnp.maximum(cnt[0, 0], 1.0)


def kernel(score, target):
    cnt, sm, loss = pl.pallas_call(
        _ce_fused_kernel,
        grid=(_B, _H // _HT),
        in_specs=[
            pl.BlockSpec((1, _C, _HT, _W), lambda b, h: (b, 0, h, 0)),
            pl.BlockSpec((1, _HT, _W), lambda b, h: (b, h, 0)),
        ],
        out_specs=[
            pl.BlockSpec((1, 1), lambda b, h: (0, 0)),
            pl.BlockSpec((1, 1), lambda b, h: (0, 0)),
            pl.BlockSpec((1, 1), lambda b, h: (0, 0)),
        ],
        out_shape=[
            jax.ShapeDtypeStruct((1, 1), jnp.float32),
            jax.ShapeDtypeStruct((1, 1), jnp.float32),
            jax.ShapeDtypeStruct((1, 1), jnp.float32),
        ],
    )(score, target)

    return lax.cond(
        cnt[0, 0] >= jnp.float32(_KEPT + 1),
        lambda ops: ops[0][0, 0],
        lambda ops: _rare_path(ops[1], ops[2]),
        (loss, score, target),
    )


# speedup vs baseline: 1.1097x; 1.1097x over previous
import functools

import jax
import jax.numpy as jnp
from jax import lax
from jax.experimental import pallas as pl
from jax.experimental.pallas import tpu as pltpu
from jax.experimental.pallas import tpu_sc as plsc

_THRESH = 0.7
_KEPT = 100000

_B, _C, _H, _W = 8, 19, 512, 512
_N = _B * _H * _W
_HT = 256
_RC = 8
_HTR = 128


_LOG2E = 1.4426950408889634
_LN2 = 0.6931471805599453
_CAP = 120.0


def _ce_fused_kernel(score_ref, target_ref, cnt_ref, sum_ref, loss_ref):
    acc_c = jnp.zeros((_RC, _W), jnp.float32)
    acc_s = jnp.zeros((_RC, _W), jnp.float32)
    for r in range(0, _HT, _RC):
        rs = pl.ds(r, _RC)
        t = target_ref[0, rs]
        y0 = jnp.minimum(score_ref[0, 0, rs] * _LOG2E, _CAP)
        y1 = jnp.minimum(score_ref[0, 1, rs] * _LOG2E, _CAP)
        s0 = jnp.exp2(y0)
        s1 = jnp.exp2(y1)
        g0 = jnp.where(t == 0, y0, 0.0)
        g1 = jnp.where(t == 1, y1, 0.0)
        for c in range(2, _C - 1, 2):
            ya = jnp.minimum(score_ref[0, c, rs] * _LOG2E, _CAP)
            yb = jnp.minimum(score_ref[0, c + 1, rs] * _LOG2E, _CAP)
            s0 = s0 + jnp.exp2(ya)
            s1 = s1 + jnp.exp2(yb)
            g0 = g0 + jnp.where(t == c, ya, 0.0)
            g1 = g1 + jnp.where(t == c + 1, yb, 0.0)
        yl = jnp.minimum(score_ref[0, _C - 1, rs] * _LOG2E, _CAP)
        s = s0 + s1 + jnp.exp2(yl)
        yt = g0 + g1 + jnp.where(t == _C - 1, yl, 0.0)
        log2p_t = yt - jnp.log2(s)
        keep = jnp.exp2(log2p_t) < _THRESH
        acc_c = acc_c + keep.astype(jnp.float32)
        acc_s = acc_s + jnp.where(keep, log2p_t, 0.0)
    cc = jnp.sum(acc_c)
    sm = jnp.sum(acc_s) * (-_LN2)

    @pl.when((pl.program_id(0) == 0) & (pl.program_id(1) == 0))
    def _():
        cnt_ref[...] = jnp.zeros((1, 1), jnp.float32)
        sum_ref[...] = jnp.zeros((1, 1), jnp.float32)

    cnt_ref[...] += cc
    sum_ref[...] += sm

    @pl.when((pl.program_id(0) == _B - 1) & (pl.program_id(1) == _H // _HT - 1))
    def _():
        loss_ref[...] = sum_ref[...] / jnp.maximum(cnt_ref[...], 1.0)


def _ce_arrays_kernel(score_ref, target_ref, pred_ref, nll_ref):
    x = score_ref[0]
    t = target_ref[0]
    m = jnp.max(x, axis=0)
    s = jnp.sum(jnp.exp(x - m[None]), axis=0)
    cls = lax.broadcasted_iota(jnp.int32, x.shape, 0)
    xt = jnp.sum(jnp.where(cls == t[None], x, 0.0), axis=0)
    logp_t = xt - m - jnp.log(s)
    pred_ref[0] = jnp.exp(logp_t)
    nll_ref[0] = -logp_t


_NW = 32
_CHUNK = _N // _NW
_SC_MESH = plsc.VectorSubcoreMesh(core_axis_name="c", subcore_axis_name="s")


def _make_sc_hist(match_shift, shift, nb):
    @functools.partial(
        pl.kernel,
        mesh=_SC_MESH,
        compiler_params=pltpu.CompilerParams(needs_layout_passes=False),
        out_type=jax.ShapeDtypeStruct((_NW, nb), jnp.int32),
        scratch_types=[
            pltpu.VMEM((_CHUNK,), jnp.float32),
            pltpu.VMEM((1, 128), jnp.int32),
            pltpu.VMEM((16, nb), jnp.int32),
            pltpu.VMEM((nb,), jnp.int32),
        ],
    )
    def sc_hist(pred_hbm, state_hbm, out_hbm, chunk_v, st_v, hist_v, loc_v):
        wid = lax.axis_index("s") * 2 + lax.axis_index("c")
        pltpu.sync_copy(pred_hbm.at[pl.ds(wid * _CHUNK, _CHUNK)], chunk_v)
        pltpu.sync_copy(state_hbm, st_v)
        pfx = st_v[0, pl.ds(0, 16)]
        zeros16 = jnp.zeros((16,), jnp.int32)

        def zero_body(j, _):
            for l in range(16):
                hist_v[l, pl.ds(j * 16, 16)] = zeros16
            return 0

        lax.fori_loop(0, nb // 16, zero_body, 0)

        lane = jnp.arange(16, dtype=jnp.int32)
        ones = jnp.ones((16,), jnp.int32)

        def hist_body(i, _):
            bits = lax.bitcast_convert_type(chunk_v[pl.ds(i * 16, 16)],
                                            jnp.int32)
            match = lax.shift_right_arithmetic(bits, match_shift) == pfx
            bin_ = lax.shift_right_arithmetic(bits, shift) & (nb - 1)
            plsc.addupdate_scatter(hist_v, [lane, bin_], ones, mask=match)
            return 0

        lax.fori_loop(0, _CHUNK // 16, hist_body, 0)

        def fold_body(j, _):
            acc = hist_v[0, pl.ds(j * 16, 16)]
            for l in range(1, 16):
                acc = acc + hist_v[l, pl.ds(j * 16, 16)]
            loc_v[pl.ds(j * 16, 16)] = acc
            return 0

        lax.fori_loop(0, nb // 16, fold_body, 0)
        pltpu.sync_copy(loc_v, out_hbm.at[wid])

    return sc_hist


def _make_locate(nb):
    def locate(hist_ref, state_ref, out_ref, thr_ref):
        tot = jnp.sum(hist_ref[...], axis=0).reshape(1, nb)
        cum = tot
        s = 1
        while s < nb:
            sh = jnp.concatenate(
                [jnp.zeros((1, s), jnp.int32), cum[:, : nb - s]], axis=1)
            cum = cum + sh
            s *= 2
        r = state_ref[0, 16]
        pfx = state_ref[0, 0]
        target = r + 1
        below = cum < target
        bstar = jnp.sum(below.astype(jnp.int32))
        cumex = jnp.sum(jnp.where(below, tot, 0))
        newpfx = pfx * nb + bstar
        newr = r - cumex
        lanes = lax.broadcasted_iota(jnp.int32, (1, 128), 1)
        out_ref[...] = jnp.where(lanes == 16, newr, newpfx)
        thr_ref[...] = lax.bitcast_convert_type(newpfx,
                                                jnp.float32).reshape(1, 1)

    return locate


def _sc_select(pred_flat):
    lanes = jnp.arange(128, dtype=jnp.int32).reshape(1, 128)
    state = jnp.where(lanes == 16, jnp.int32(_KEPT), 0)
    thr = None
    for match_shift, shift, nb in ((31, 20, 2048), (20, 10, 1024),
                                   (10, 0, 1024)):
        hists = _make_sc_hist(match_shift, shift, nb)(pred_flat, state)
        state, thr = pl.pallas_call(
            _make_locate(nb),
            in_specs=[
                pl.BlockSpec((_NW, nb), lambda: (0, 0)),
                pl.BlockSpec((1, 128), lambda: (0, 0)),
            ],
            out_specs=[
                pl.BlockSpec((1, 128), lambda: (0, 0)),
                pl.BlockSpec((1, 1), lambda: (0, 0)),
            ],
            out_shape=[
                jax.ShapeDtypeStruct((1, 128), jnp.int32),
                jax.ShapeDtypeStruct((1, 1), jnp.float32),
            ],
        )(hists, state)
    return thr


def _masked_sum_kernel(pred_ref, nll_ref, thr_ref, cnt_ref, sum_ref):
    thr = thr_ref[0, 0]
    keep = pred_ref[...] < thr
    c = jnp.sum(keep.astype(jnp.float32))
    sm = jnp.sum(jnp.where(keep, nll_ref[...], 0.0))

    @pl.when(pl.program_id(0) == 0)
    def _():
        cnt_ref[...] = jnp.zeros((1, 1), jnp.float32)
        sum_ref[...] = jnp.zeros((1, 1), jnp.float32)

    cnt_ref[...] += c
    sum_ref[...] += sm


def _rare_path(score, target):
    pred, nll = pl.pallas_call(
        _ce_arrays_kernel,
        grid=(_B, _H // _HTR),
        in_specs=[
            pl.BlockSpec((1, _C, _HTR, _W), lambda b, h: (b, 0, h, 0)),
            pl.BlockSpec((1, _HTR, _W), lambda b, h: (b, h, 0)),
        ],
        out_specs=[
            pl.BlockSpec((1, _HTR, _W), lambda b, h: (b, h, 0)),
            pl.BlockSpec((1, _HTR, _W), lambda b, h: (b, h, 0)),
        ],
        out_shape=[
            jax.ShapeDtypeStruct((_B, _H, _W), jnp.float32),
            jax.ShapeDtypeStruct((_B, _H, _W), jnp.float32),
        ],
    )(score, target)
    pred2 = pred.reshape(_N // 1024, 1024)
    nll2 = nll.reshape(_N // 1024, 1024)

    thr = _sc_select(pred.reshape(_N))

    rows = _N // 1024
    rt = rows // 8
    cnt, sm = pl.pallas_call(
        _masked_sum_kernel,
        grid=(8,),
        in_specs=[
            pl.BlockSpec((rt, 1024), lambda i: (i, 0)),
            pl.BlockSpec((rt, 1024), lambda i: (i, 0)),
            pl.BlockSpec((1, 1), lambda i: (0, 0)),
        ],
        out_specs=[
            pl.BlockSpec((1, 1), lambda i: (0, 0)),
            pl.BlockSpec((1, 1), lambda i: (0, 0)),
        ],
        out_shape=[
            jax.ShapeDtypeStruct((1, 1), jnp.float32),
            jax.ShapeDtypeStruct((1, 1), jnp.float32),
        ],
    )(pred2, nll2, thr)
    return sm[0, 0] / jnp.maximum(cnt[0, 0], 1.0)


def kernel(score, target):
    cnt, sm, loss = pl.pallas_call(
        _ce_fused_kernel,
        grid=(_B, _H // _HT),
        in_specs=[
            pl.BlockSpec((1, _C, _HT, _W), lambda b, h: (b, 0, h, 0)),
            pl.BlockSpec((1, _HT, _W), lambda b, h: (b, h, 0)),
        ],
        out_specs=[
            pl.BlockSpec((1, 1), lambda b, h: (0, 0)),
            pl.BlockSpec((1, 1), lambda b, h: (0, 0)),
            pl.BlockSpec((1, 1), lambda b, h: (0, 0)),
        ],
        out_shape=[
            jax.ShapeDtypeStruct((1, 1), jnp.float32),
            jax.ShapeDtypeStruct((1, 1), jnp.float32),
            jax.ShapeDtypeStruct((1, 1), jnp.float32),
        ],
    )(score, target)

    return lax.cond(
        cnt[0, 0] >= jnp.float32(_KEPT + 1),
        lambda ops: ops[0][0, 0],
        lambda ops: _rare_path(ops[1], ops[2]),
        (loss, score, target),
    )


# speedup vs baseline: 1.1397x; 1.0270x over previous
import functools

import jax
import jax.numpy as jnp
from jax import lax
from jax.experimental import pallas as pl
from jax.experimental.pallas import tpu as pltpu
from jax.experimental.pallas import tpu_sc as plsc

_THRESH = 0.7
_KEPT = 100000

_B, _C, _H, _W = 8, 19, 512, 512
_N = _B * _H * _W
_HT = 512
_RC = 8
_HTR = 128


_LOG2E = 1.4426950408889634
_LN2 = 0.6931471805599453
_CAP = 120.0


def _ce_fused_kernel(score_ref, target_ref, cnt_ref, sum_ref, loss_ref):
    acc_c = jnp.zeros((_RC, _W), jnp.float32)
    acc_s = jnp.zeros((_RC, _W), jnp.float32)
    for r in range(0, _HT, _RC):
        rs = pl.ds(r, _RC)
        t = target_ref[0, rs]
        y0 = jnp.minimum(score_ref[0, 0, rs] * _LOG2E, _CAP)
        y1 = jnp.minimum(score_ref[0, 1, rs] * _LOG2E, _CAP)
        s0 = jnp.exp2(y0)
        s1 = jnp.exp2(y1)
        g0 = jnp.where(t == 0, y0, 0.0)
        g1 = jnp.where(t == 1, y1, 0.0)
        for c in range(2, _C - 1, 2):
            ya = jnp.minimum(score_ref[0, c, rs] * _LOG2E, _CAP)
            yb = jnp.minimum(score_ref[0, c + 1, rs] * _LOG2E, _CAP)
            s0 = s0 + jnp.exp2(ya)
            s1 = s1 + jnp.exp2(yb)
            g0 = g0 + jnp.where(t == c, ya, 0.0)
            g1 = g1 + jnp.where(t == c + 1, yb, 0.0)
        yl = jnp.minimum(score_ref[0, _C - 1, rs] * _LOG2E, _CAP)
        s = s0 + s1 + jnp.exp2(yl)
        yt = g0 + g1 + jnp.where(t == _C - 1, yl, 0.0)
        log2p_t = yt - jnp.log2(s)
        keep = jnp.exp2(log2p_t) < _THRESH
        acc_c = acc_c + keep.astype(jnp.float32)
        acc_s = acc_s + jnp.where(keep, log2p_t, 0.0)
    cc = jnp.sum(acc_c)
    sm = jnp.sum(acc_s) * (-_LN2)

    @pl.when((pl.program_id(0) == 0) & (pl.program_id(1) == 0))
    def _():
        cnt_ref[...] = jnp.zeros((1, 1), jnp.float32)
        sum_ref[...] = jnp.zeros((1, 1), jnp.float32)

    cnt_ref[...] += cc
    sum_ref[...] += sm

    @pl.when((pl.program_id(0) == _B - 1) & (pl.program_id(1) == _H // _HT - 1))
    def _():
        loss_ref[...] = sum_ref[...] / jnp.maximum(cnt_ref[...], 1.0)


def _ce_arrays_kernel(score_ref, target_ref, pred_ref, nll_ref):
    x = score_ref[0]
    t = target_ref[0]
    m = jnp.max(x, axis=0)
    s = jnp.sum(jnp.exp(x - m[None]), axis=0)
    cls = lax.broadcasted_iota(jnp.int32, x.shape, 0)
    xt = jnp.sum(jnp.where(cls == t[None], x, 0.0), axis=0)
    logp_t = xt - m - jnp.log(s)
    pred_ref[0] = jnp.exp(logp_t)
    nll_ref[0] = -logp_t


_NW = 32
_CHUNK = _N // _NW
_SC_MESH = plsc.VectorSubcoreMesh(core_axis_name="c", subcore_axis_name="s")


def _make_sc_hist(match_shift, shift, nb):
    @functools.partial(
        pl.kernel,
        mesh=_SC_MESH,
        compiler_params=pltpu.CompilerParams(needs_layout_passes=False),
        out_type=jax.ShapeDtypeStruct((_NW, nb), jnp.int32),
        scratch_types=[
            pltpu.VMEM((_CHUNK,), jnp.float32),
            pltpu.VMEM((1, 128), jnp.int32),
            pltpu.VMEM((16, nb), jnp.int32),
            pltpu.VMEM((nb,), jnp.int32),
        ],
    )
    def sc_hist(pred_hbm, state_hbm, out_hbm, chunk_v, st_v, hist_v, loc_v):
        wid = lax.axis_index("s") * 2 + lax.axis_index("c")
        pltpu.sync_copy(pred_hbm.at[pl.ds(wid * _CHUNK, _CHUNK)], chunk_v)
        pltpu.sync_copy(state_hbm, st_v)
        pfx = st_v[0, pl.ds(0, 16)]
        zeros16 = jnp.zeros((16,), jnp.int32)

        def zero_body(j, _):
            for l in range(16):
                hist_v[l, pl.ds(j * 16, 16)] = zeros16
            return 0

        lax.fori_loop(0, nb // 16, zero_body, 0)

        lane = jnp.arange(16, dtype=jnp.int32)
        ones = jnp.ones((16,), jnp.int32)

        def hist_body(i, _):
            bits = lax.bitcast_convert_type(chunk_v[pl.ds(i * 16, 16)],
                                            jnp.int32)
            match = lax.shift_right_arithmetic(bits, match_shift) == pfx
            bin_ = lax.shift_right_arithmetic(bits, shift) & (nb - 1)
            plsc.addupdate_scatter(hist_v, [lane, bin_], ones, mask=match)
            return 0

        lax.fori_loop(0, _CHUNK // 16, hist_body, 0)

        def fold_body(j, _):
            acc = hist_v[0, pl.ds(j * 16, 16)]
            for l in range(1, 16):
                acc = acc + hist_v[l, pl.ds(j * 16, 16)]
            loc_v[pl.ds(j * 16, 16)] = acc
            return 0

        lax.fori_loop(0, nb // 16, fold_body, 0)
        pltpu.sync_copy(loc_v, out_hbm.at[wid])

    return sc_hist


def _make_locate(nb):
    def locate(hist_ref, state_ref, out_ref, thr_ref):
        tot = jnp.sum(hist_ref[...], axis=0).reshape(1, nb)
        cum = tot
        s = 1
        while s < nb:
            sh = jnp.concatenate(
                [jnp.zeros((1, s), jnp.int32), cum[:, : nb - s]], axis=1)
            cum = cum + sh
            s *= 2
        r = state_ref[0, 16]
        pfx = state_ref[0, 0]
        target = r + 1
        below = cum < target
        bstar = jnp.sum(below.astype(jnp.int32))
        cumex = jnp.sum(jnp.where(below, tot, 0))
        newpfx = pfx * nb + bstar
        newr = r - cumex
        lanes = lax.broadcasted_iota(jnp.int32, (1, 128), 1)
        out_ref[...] = jnp.where(lanes == 16, newr, newpfx)
        thr_ref[...] = lax.bitcast_convert_type(newpfx,
                                                jnp.float32).reshape(1, 1)

    return locate


def _sc_select(pred_flat):
    lanes = jnp.arange(128, dtype=jnp.int32).reshape(1, 128)
    state = jnp.where(lanes == 16, jnp.int32(_KEPT), 0)
    thr = None
    for match_shift, shift, nb in ((31, 20, 2048), (20, 10, 1024),
                                   (10, 0, 1024)):
        hists = _make_sc_hist(match_shift, shift, nb)(pred_flat, state)
        state, thr = pl.pallas_call(
            _make_locate(nb),
            in_specs=[
                pl.BlockSpec((_NW, nb), lambda: (0, 0)),
                pl.BlockSpec((1, 128), lambda: (0, 0)),
            ],
            out_specs=[
                pl.BlockSpec((1, 128), lambda: (0, 0)),
                pl.BlockSpec((1, 1), lambda: (0, 0)),
            ],
            out_shape=[
                jax.ShapeDtypeStruct((1, 128), jnp.int32),
                jax.ShapeDtypeStruct((1, 1), jnp.float32),
            ],
        )(hists, state)
    return thr


def _masked_sum_kernel(pred_ref, nll_ref, thr_ref, cnt_ref, sum_ref):
    thr = thr_ref[0, 0]
    keep = pred_ref[...] < thr
    c = jnp.sum(keep.astype(jnp.float32))
    sm = jnp.sum(jnp.where(keep, nll_ref[...], 0.0))

    @pl.when(pl.program_id(0) == 0)
    def _():
        cnt_ref[...] = jnp.zeros((1, 1), jnp.float32)
        sum_ref[...] = jnp.zeros((1, 1), jnp.float32)

    cnt_ref[...] += c
    sum_ref[...] += sm


def _rare_path(score, target):
    pred, nll = pl.pallas_call(
        _ce_arrays_kernel,
        grid=(_B, _H // _HTR),
        in_specs=[
            pl.BlockSpec((1, _C, _HTR, _W), lambda b, h: (b, 0, h, 0)),
            pl.BlockSpec((1, _HTR, _W), lambda b, h: (b, h, 0)),
        ],
        out_specs=[
            pl.BlockSpec((1, _HTR, _W), lambda b, h: (b, h, 0)),
            pl.BlockSpec((1, _HTR, _W), lambda b, h: (b, h, 0)),
        ],
        out_shape=[
            jax.ShapeDtypeStruct((_B, _H, _W), jnp.float32),
            jax.ShapeDtypeStruct((_B, _H, _W), jnp.float32),
        ],
    )(score, target)
    pred2 = pred.reshape(_N // 1024, 1024)
    nll2 = nll.reshape(_N // 1024, 1024)

    thr = _sc_select(pred.reshape(_N))

    rows = _N // 1024
    rt = rows // 8
    cnt, sm = pl.pallas_call(
        _masked_sum_kernel,
        grid=(8,),
        in_specs=[
            pl.BlockSpec((rt, 1024), lambda i: (i, 0)),
            pl.BlockSpec((rt, 1024), lambda i: (i, 0)),
            pl.BlockSpec((1, 1), lambda i: (0, 0)),
        ],
        out_specs=[
            pl.BlockSpec((1, 1), lambda i: (0, 0)),
            pl.BlockSpec((1, 1), lambda i: (0, 0)),
        ],
        out_shape=[
            jax.ShapeDtypeStruct((1, 1), jnp.float32),
            jax.ShapeDtypeStruct((1, 1), jnp.float32),
        ],
    )(pred2, nll2, thr)
    return sm[0, 0] / jnp.maximum(cnt[0, 0], 1.0)


def kernel(score, target):
    cnt, sm, loss = pl.pallas_call(
        _ce_fused_kernel,
        grid=(_B, _H // _HT),
        in_specs=[
            pl.BlockSpec((1, _C, _HT, _W), lambda b, h: (b, 0, h, 0)),
            pl.BlockSpec((1, _HT, _W), lambda b, h: (b, h, 0)),
        ],
        out_specs=[
            pl.BlockSpec((1, 1), lambda b, h: (0, 0)),
            pl.BlockSpec((1, 1), lambda b, h: (0, 0)),
            pl.BlockSpec((1, 1), lambda b, h: (0, 0)),
        ],
        out_shape=[
            jax.ShapeDtypeStruct((1, 1), jnp.float32),
            jax.ShapeDtypeStruct((1, 1), jnp.float32),
            jax.ShapeDtypeStruct((1, 1), jnp.float32),
        ],
    )(score, target)

    return lax.cond(
        cnt[0, 0] >= jnp.float32(_KEPT + 1),
        lambda ops: ops[0][0, 0],
        lambda ops: _rare_path(ops[1], ops[2]),
        (loss, score, target),
    )
